# Initial kernel scaffold; baseline (speedup 1.0000x reference)
import jax
import jax.numpy as jnp
from jax.experimental import pallas as pl


def kernel(node_features, edge_index, edge_logits, edge_params):
    raise NotImplementedError("write your pallas kernel here")


# speedup vs baseline: 17.7103x; 17.7103x over previous
import functools

import jax
import jax.numpy as jnp
from jax import lax
from jax.experimental import pallas as pl
from jax.experimental.pallas import tpu as pltpu
from jax.experimental.pallas import tpu_sc as plsc

N_NODES = 10000
N_EDGES = 160000
LANES = 16
NUM_CORES = 2
NUM_SUBCORES = 16
NUM_WORKERS = NUM_CORES * NUM_SUBCORES
CHUNK = N_EDGES // NUM_WORKERS
NFULL = CHUNK // LANES
ACC_ROWS = 640
ROW_SUM_R, ROW_SUM_X, ROW_SQ_R, ROW_SQ_X = 625, 626, 627, 628


def _sc_body(v_hbm, ei_hbm, lg_hbm, par_hbm, out_hbm,
             v_v, src_v, dst_v, lg_v, par_v, acc_v,
             sem0, sem1, sem2, sem3, sem4):
    c = lax.axis_index("c")
    s = lax.axis_index("s")
    w = c * NUM_SUBCORES + s
    ebase = w * CHUNK

    cps = [
        pltpu.async_copy(v_hbm, v_v, sem0),
        pltpu.async_copy(ei_hbm.at[pl.ds(ebase, CHUNK)], src_v, sem1),
        pltpu.async_copy(ei_hbm.at[pl.ds(N_EDGES + ebase, CHUNK)], dst_v, sem2),
        pltpu.async_copy(lg_hbm.at[pl.ds(ebase, CHUNK)], lg_v, sem3),
        pltpu.async_copy(par_hbm.at[pl.ds(2 * ebase, 2 * CHUNK)], par_v, sem4),
    ]

    lane = lax.iota(jnp.int32, LANES)
    zero16 = jnp.zeros((LANES,), jnp.float32)

    def _zero(i, _):
        acc_v[i, :] = zero16
        return 0
    lax.fori_loop(0, ACC_ROWS, _zero, 0, unroll=8)
    for cp in cps:
        cp.wait()

    def _edge_step(b, eidx, carry, mask):
        s_r, s_x, q_r, q_x = carry
        si = src_v[pl.ds(b, LANES)]
        di = dst_v[pl.ds(b, LANES)]
        lg = lg_v[pl.ds(b, LANES)]
        if mask is not None:
            si = jnp.where(mask, si, 0)
            di = jnp.where(mask, di, 0)
            lg = jnp.where(mask, lg, jnp.float32(0.0))
        vs = plsc.load_gather(v_v, [si])
        vd = plsc.load_gather(v_v, [di])
        rr = plsc.load_gather(par_v, [2 * eidx])
        rx = plsc.load_gather(par_v, [2 * eidx + 1])
        if mask is not None:
            rr = jnp.where(mask, rr, jnp.float32(0.0))
            rx = jnp.where(mask, rx, jnp.float32(0.0))
        imp = rr + rx + jnp.float32(1e-6)
        diff = jnp.abs(vs - vd)
        wgt = diff / (imp * (jnp.float32(1.0) + jnp.exp(-lg)))
        plsc.addupdate_scatter(
            acc_v, [lax.shift_right_logical(di, 4), di & 15], wgt)
        plsc.addupdate_scatter(
            acc_v, [lax.shift_right_logical(si, 4), si & 15], -wgt)
        return (s_r + rr, s_x + rx, q_r + rr * rr, q_x + rx * rx)

    def _full_step(i, carry):
        return _edge_step(i * LANES, i * LANES + lane, carry, None)

    carry = lax.fori_loop(
        0, NFULL, _full_step, (zero16, zero16, zero16, zero16))
    tb = CHUNK - LANES
    s_r, s_x, q_r, q_x = _edge_step(tb, tb + lane, carry, lane >= 8)

    acc_v[ROW_SUM_R, :] = s_r
    acc_v[ROW_SUM_X, :] = s_x
    acc_v[ROW_SQ_R, :] = q_r
    acc_v[ROW_SQ_X, :] = q_x

    pltpu.sync_copy(acc_v, out_hbm.at[w])


def _make_sc_kernel(interpret=False):
    return functools.partial(
        pl.kernel,
        out_type=jax.ShapeDtypeStruct(
            (NUM_WORKERS, ACC_ROWS, LANES), jnp.float32),
        mesh=plsc.VectorSubcoreMesh(core_axis_name="c", subcore_axis_name="s"),
        compiler_params=pltpu.CompilerParams(needs_layout_passes=False),
        interpret=interpret,
        scratch_types=[
            pltpu.VMEM((N_NODES,), jnp.float32),
            pltpu.VMEM((CHUNK,), jnp.int32),
            pltpu.VMEM((CHUNK,), jnp.int32),
            pltpu.VMEM((CHUNK,), jnp.float32),
            pltpu.VMEM((2 * CHUNK,), jnp.float32),
            pltpu.VMEM((ACC_ROWS, LANES), jnp.float32),
            pltpu.SemaphoreType.DMA,
            pltpu.SemaphoreType.DMA,
            pltpu.SemaphoreType.DMA,
            pltpu.SemaphoreType.DMA,
            pltpu.SemaphoreType.DMA,
        ],
    )(_sc_body)


_sc_kernel = _make_sc_kernel()


def _finish_kernel(p_ref, o_ref):
    p = p_ref[...]
    tot = jnp.sum(p, axis=0)
    row = lax.broadcasted_iota(jnp.int32, (ACC_ROWS, LANES), 0)
    nodes = jnp.where(row < 625, tot, 0.0)
    kcl = jnp.sum(nodes * nodes) / jnp.float32(N_NODES)
    s_r = jnp.sum(jnp.where(row == ROW_SUM_R, tot, 0.0))
    s_x = jnp.sum(jnp.where(row == ROW_SUM_X, tot, 0.0))
    q_r = jnp.sum(jnp.where(row == ROW_SQ_R, tot, 0.0))
    q_x = jnp.sum(jnp.where(row == ROW_SQ_X, tot, 0.0))
    n = jnp.float32(N_EDGES)
    var_r = (q_r - s_r * s_r / n) / (n - 1.0)
    var_x = (q_x - s_x * s_x / n) / (n - 1.0)
    o_ref[0, 0] = kcl + 0.5 * (var_r + var_x)


def kernel(node_features, edge_index, edge_logits, edge_params):
    v = node_features[:, 0]
    partial = _sc_kernel(v, edge_index.reshape(-1), edge_logits,
                         edge_params.reshape(-1))
    out = pl.pallas_call(
        _finish_kernel,
        out_shape=jax.ShapeDtypeStruct((1, 1), jnp.float32),
        out_specs=pl.BlockSpec(memory_space=pltpu.SMEM),
    )(partial)
    return out[0, 0]


# speedup vs baseline: 44.4337x; 2.5089x over previous
import functools

import jax
import jax.numpy as jnp
from jax import lax
from jax.experimental import pallas as pl
from jax.experimental.pallas import tpu as pltpu
from jax.experimental.pallas import tpu_sc as plsc

N_NODES = 10000
N_EDGES = 160000
D_FEAT = 128
LANES = 16
NUM_CORES = 2
NUM_SUBCORES = 16
NUM_WORKERS = NUM_CORES * NUM_SUBCORES
CHUNK = N_EDGES // NUM_WORKERS
NFULL = CHUNK // LANES
ACC_ROWS = 640


def _sc_body(nf_hbm, ei_hbm, lg_hbm, r_hbm, x_hbm, out_hbm,
             vidx_v, v_v, src_v, dst_v, lg_v, r_v, x_v, acc_v,
             sem0, sem1, sem2, sem3, sem4, sem5):
    c = lax.axis_index("c")
    s = lax.axis_index("s")
    w = c * NUM_SUBCORES + s
    ebase = w * CHUNK

    cps = [
        pltpu.async_copy(ei_hbm.at[pl.ds(ebase, CHUNK)], src_v, sem0),
        pltpu.async_copy(ei_hbm.at[pl.ds(N_EDGES + ebase, CHUNK)], dst_v,
                         sem1),
        pltpu.async_copy(lg_hbm.at[pl.ds(ebase, CHUNK)], lg_v, sem2),
        pltpu.async_copy(r_hbm.at[pl.ds(ebase, CHUNK)], r_v, sem3),
        pltpu.async_copy(x_hbm.at[pl.ds(ebase, CHUNK)], x_v, sem4),
    ]

    lane = lax.iota(jnp.int32, LANES)
    zero16 = jnp.zeros((LANES,), jnp.float32)

    lane128 = lane * D_FEAT

    def _fill(i, _):
        vidx_v[pl.ds(i * LANES, LANES)] = lane128 + i * (LANES * D_FEAT)
        return 0
    lax.fori_loop(0, N_NODES // LANES, _fill, 0, unroll=8)
    cps.append(pltpu.async_copy(nf_hbm.at[vidx_v], v_v, sem5))

    def _zero(i, _):
        acc_v[i, :] = zero16
        return 0
    lax.fori_loop(0, ACC_ROWS, _zero, 0, unroll=8)
    for cp in cps:
        cp.wait()

    def _edge_step(b, mask):
        si = src_v[pl.ds(b, LANES)]
        di = dst_v[pl.ds(b, LANES)]
        if mask is not None:
            si = jnp.where(mask, si, 0)
            di = jnp.where(mask, di, 0)
        lg = lg_v[pl.ds(b, LANES)]
        rr = r_v[pl.ds(b, LANES)]
        rx = x_v[pl.ds(b, LANES)]
        vs = plsc.load_gather(v_v, [si])
        vd = plsc.load_gather(v_v, [di])
        imp = rr + rx + jnp.float32(1e-6)
        diff = jnp.abs(vs - vd)
        wgt = diff / (imp * (jnp.float32(1.0) + jnp.exp(-lg)))
        plsc.addupdate_scatter(
            acc_v, [lax.shift_right_logical(di, 4), di & 15], wgt)
        plsc.addupdate_scatter(
            acc_v, [lax.shift_right_logical(si, 4), si & 15], -wgt)

    plsc.parallel_loop(0, NFULL, unroll=4)(
        lambda i: _edge_step(i * LANES, None))
    _edge_step(CHUNK - LANES, lane >= 8)

    pltpu.sync_copy(acc_v, out_hbm.at[w])


def _make_sc_kernel(interpret=False):
    return functools.partial(
        pl.kernel,
        out_type=jax.ShapeDtypeStruct(
            (NUM_WORKERS, ACC_ROWS, LANES), jnp.float32),
        mesh=plsc.VectorSubcoreMesh(core_axis_name="c", subcore_axis_name="s"),
        compiler_params=pltpu.CompilerParams(needs_layout_passes=False),
        interpret=interpret,
        scratch_types=[
            pltpu.VMEM((N_NODES,), jnp.int32),
            pltpu.VMEM((N_NODES,), jnp.float32),
            pltpu.VMEM((CHUNK,), jnp.int32),
            pltpu.VMEM((CHUNK,), jnp.int32),
            pltpu.VMEM((CHUNK,), jnp.float32),
            pltpu.VMEM((CHUNK,), jnp.float32),
            pltpu.VMEM((CHUNK,), jnp.float32),
            pltpu.VMEM((ACC_ROWS, LANES), jnp.float32),
            pltpu.SemaphoreType.DMA,
            pltpu.SemaphoreType.DMA,
            pltpu.SemaphoreType.DMA,
            pltpu.SemaphoreType.DMA,
            pltpu.SemaphoreType.DMA,
            pltpu.SemaphoreType.DMA,
        ],
    )(_sc_body)


_sc_kernel = _make_sc_kernel()


def _finish_kernel(p_ref, r_ref, x_ref, o_ref):
    tot = jnp.sum(p_ref[...], axis=0)
    kcl = jnp.sum(tot * tot) / jnp.float32(N_NODES)
    rv = r_ref[...]
    xv = x_ref[...]
    n = jnp.float32(N_EDGES)
    var_r = (jnp.sum(rv * rv) - jnp.sum(rv) ** 2 / n) / (n - 1.0)
    var_x = (jnp.sum(xv * xv) - jnp.sum(xv) ** 2 / n) / (n - 1.0)
    o_ref[0, 0] = kcl + 0.5 * (var_r + var_x)


def kernel(node_features, edge_index, edge_logits, edge_params):
    nf_flat = node_features.reshape(-1)
    r = edge_params[:, 0]
    x = edge_params[:, 1]
    partial = _sc_kernel(nf_flat, edge_index.reshape(-1), edge_logits, r, x)
    out = pl.pallas_call(
        _finish_kernel,
        out_shape=jax.ShapeDtypeStruct((1, 1), jnp.float32),
        out_specs=pl.BlockSpec(memory_space=pltpu.SMEM),
    )(partial, r, x)
    return out[0, 0]


# speedup vs baseline: 51.5122x; 1.1593x over previous
import functools

import jax
import jax.numpy as jnp
from jax import lax
from jax.experimental import pallas as pl
from jax.experimental.pallas import tpu as pltpu
from jax.experimental.pallas import tpu_sc as plsc

N_NODES = 10000
N_EDGES = 160000
D_FEAT = 128
LANES = 16
NUM_CORES = 2
NUM_SUBCORES = 16
NUM_WORKERS = NUM_CORES * NUM_SUBCORES
CHUNK = N_EDGES // NUM_WORKERS
NFULL = CHUNK // LANES
ACC_WORDS = 10240
OFF_SUM_R, OFF_SUM_X, OFF_SQ_R, OFF_SQ_X = 10000, 10016, 10032, 10048
VRANGE = 640
STRIPE = ACC_WORDS // NUM_SUBCORES


def _sc_body(nf_hbm, ei_hbm, lg_hbm, r_hbm, x_hbm, out_hbm,
             ring0_v, ring1_v, ring2_v, ring3_v, vtmp_v, v_v,
             src_v, dst_v, lg_v, r_v, x_v,
             acc_v, tmp_v, red_v, shv_s, shacc_s,
             sem0, sem1, sem2, sem3, sem4, rsem0, rsem1, rsem2, rsem3):
    c = lax.axis_index("c")
    s = lax.axis_index("s")
    w = c * NUM_SUBCORES + s
    ebase = w * CHUNK

    cps = [
        pltpu.async_copy(ei_hbm.at[pl.ds(ebase, CHUNK)], src_v, sem0),
        pltpu.async_copy(ei_hbm.at[pl.ds(N_EDGES + ebase, CHUNK)], dst_v,
                         sem1),
        pltpu.async_copy(lg_hbm.at[pl.ds(ebase, CHUNK)], lg_v, sem2),
        pltpu.async_copy(r_hbm.at[pl.ds(ebase, CHUNK)], r_v, sem3),
        pltpu.async_copy(x_hbm.at[pl.ds(ebase, CHUNK)], x_v, sem4),
    ]

    lane = lax.iota(jnp.int32, LANES)
    zero16 = jnp.zeros((LANES,), jnp.float32)

    vstart = jnp.minimum(VRANGE * s, N_NODES - VRANGE)
    rings = [ring0_v, ring1_v, ring2_v, ring3_v]
    rsems = [rsem0, rsem1, rsem2, rsem3]
    lane128 = lane * D_FEAT
    nslab = VRANGE // LANES

    def _issue(j):
        return pltpu.async_copy(
            nf_hbm.at[pl.ds((vstart + LANES * j) * D_FEAT, LANES * D_FEAT)],
            rings[j % 4], rsems[j % 4])

    ring_cps = [_issue(j) for j in range(4)]
    for j in range(nslab):
        ring_cps[j % 4].wait()
        vals = plsc.load_gather(rings[j % 4], [lane128])
        if j + 4 < nslab:
            ring_cps[j % 4] = _issue(j + 4)
        vtmp_v[pl.ds(LANES * j, LANES)] = vals
    pltpu.sync_copy(vtmp_v, shv_s.at[pl.ds(vstart, VRANGE)])

    def _zero(i, _):
        acc_v[pl.ds(i * LANES, LANES)] = zero16
        return 0
    lax.fori_loop(0, ACC_WORDS // LANES, _zero, 0, unroll=8)
    plsc.subcore_barrier()
    pltpu.sync_copy(shv_s, v_v)
    for cp in cps:
        cp.wait()

    def _edge_step(b, mask):
        si = src_v[pl.ds(b, LANES)]
        di = dst_v[pl.ds(b, LANES)]
        if mask is not None:
            si = jnp.where(mask, si, 0)
            di = jnp.where(mask, di, 0)
        lg = lg_v[pl.ds(b, LANES)]
        rr = r_v[pl.ds(b, LANES)]
        rx = x_v[pl.ds(b, LANES)]
        vs = plsc.load_gather(v_v, [si])
        vd = plsc.load_gather(v_v, [di])
        imp = rr + rx + jnp.float32(1e-6)
        diff = jnp.abs(vs - vd)
        wgt = diff / (imp * (jnp.float32(1.0) + jnp.exp(-lg)))
        plsc.addupdate_scatter(acc_v, [di], wgt)
        plsc.addupdate_scatter(acc_v, [si], -wgt)

    plsc.parallel_loop(0, NFULL, unroll=4)(
        lambda i: _edge_step(i * LANES, None))
    _edge_step(CHUNK - LANES, lane >= 8)

    def _kvl(i, carry):
        s_r, s_x, q_r, q_x = carry
        rr = r_v[pl.ds(i * LANES, LANES)]
        rx = x_v[pl.ds(i * LANES, LANES)]
        return (s_r + rr, s_x + rx, q_r + rr * rr, q_x + rx * rx)

    s_r, s_x, q_r, q_x = lax.fori_loop(
        0, NFULL, _kvl, (zero16, zero16, zero16, zero16), unroll=4)
    tmask = lane >= 8
    rr = jnp.where(tmask, r_v[pl.ds(CHUNK - LANES, LANES)], 0.0)
    rx = jnp.where(tmask, x_v[pl.ds(CHUNK - LANES, LANES)], 0.0)
    acc_v[pl.ds(OFF_SUM_R, LANES)] = s_r + rr
    acc_v[pl.ds(OFF_SUM_X, LANES)] = s_x + rx
    acc_v[pl.ds(OFF_SQ_R, LANES)] = q_r + rr * rr
    acc_v[pl.ds(OFF_SQ_X, LANES)] = q_x + rx * rx

    pltpu.sync_copy(acc_v, shacc_s.at[s])
    plsc.subcore_barrier()
    words = pl.ds(STRIPE * s, STRIPE)
    pltpu.sync_copy(shacc_s.at[0, words], red_v)
    for j in range(1, NUM_SUBCORES):
        pltpu.sync_copy(shacc_s.at[j, words], tmp_v)

        def _acc_vec(k, _):
            sl = pl.ds(k * LANES, LANES)
            red_v[sl] = red_v[sl] + tmp_v[sl]
            return 0
        lax.fori_loop(0, STRIPE // LANES, _acc_vec, 0, unroll=8)
    pltpu.sync_copy(red_v, out_hbm.at[c, words])


def _make_sc_kernel(interpret=False):
    return functools.partial(
        pl.kernel,
        out_type=jax.ShapeDtypeStruct((NUM_CORES, ACC_WORDS), jnp.float32),
        mesh=plsc.VectorSubcoreMesh(core_axis_name="c", subcore_axis_name="s"),
        compiler_params=pltpu.CompilerParams(needs_layout_passes=False),
        interpret=interpret,
        scratch_types=[
            pltpu.VMEM((LANES * D_FEAT,), jnp.float32),
            pltpu.VMEM((LANES * D_FEAT,), jnp.float32),
            pltpu.VMEM((LANES * D_FEAT,), jnp.float32),
            pltpu.VMEM((LANES * D_FEAT,), jnp.float32),
            pltpu.VMEM((VRANGE,), jnp.float32),
            pltpu.VMEM((N_NODES,), jnp.float32),
            pltpu.VMEM((CHUNK,), jnp.int32),
            pltpu.VMEM((CHUNK,), jnp.int32),
            pltpu.VMEM((CHUNK,), jnp.float32),
            pltpu.VMEM((CHUNK,), jnp.float32),
            pltpu.VMEM((CHUNK,), jnp.float32),
            pltpu.VMEM((ACC_WORDS,), jnp.float32),
            pltpu.VMEM((STRIPE,), jnp.float32),
            pltpu.VMEM((STRIPE,), jnp.float32),
            pltpu.VMEM_SHARED((N_NODES,), jnp.float32),
            pltpu.VMEM_SHARED((NUM_SUBCORES, ACC_WORDS), jnp.float32),
            pltpu.SemaphoreType.DMA,
            pltpu.SemaphoreType.DMA,
            pltpu.SemaphoreType.DMA,
            pltpu.SemaphoreType.DMA,
            pltpu.SemaphoreType.DMA,
            pltpu.SemaphoreType.DMA,
            pltpu.SemaphoreType.DMA,
            pltpu.SemaphoreType.DMA,
            pltpu.SemaphoreType.DMA,
        ],
    )(_sc_body)


_sc_kernel = _make_sc_kernel()


def _finish_kernel(p_ref, o_ref):
    p = p_ref[...]
    tot = (p[0] + p[1]).reshape(ACC_WORDS // D_FEAT, D_FEAT)
    fidx = (lax.broadcasted_iota(jnp.int32, tot.shape, 0) * D_FEAT
            + lax.broadcasted_iota(jnp.int32, tot.shape, 1))
    nodes = jnp.where(fidx < N_NODES, tot, 0.0)
    kcl = jnp.sum(nodes * nodes) / jnp.float32(N_NODES)

    def _sum16(off):
        m = (fidx >= off) & (fidx < off + LANES)
        return jnp.sum(jnp.where(m, tot, 0.0))

    s_r, s_x = _sum16(OFF_SUM_R), _sum16(OFF_SUM_X)
    q_r, q_x = _sum16(OFF_SQ_R), _sum16(OFF_SQ_X)
    n = jnp.float32(N_EDGES)
    var_r = (q_r - s_r * s_r / n) / (n - 1.0)
    var_x = (q_x - s_x * s_x / n) / (n - 1.0)
    o_ref[0, 0] = kcl + 0.5 * (var_r + var_x)


def kernel(node_features, edge_index, edge_logits, edge_params):
    nf_flat = node_features.reshape(-1)
    r = edge_params[:, 0]
    x = edge_params[:, 1]
    partial = _sc_kernel(nf_flat, edge_index.reshape(-1), edge_logits, r, x)
    out = pl.pallas_call(
        _finish_kernel,
        out_shape=jax.ShapeDtypeStruct((1, 1), jnp.float32),
        out_specs=pl.BlockSpec(memory_space=pltpu.SMEM),
    )(partial)
    return out[0, 0]
